# baseline (speedup 1.0000x reference)
import jax
import jax.numpy as jnp
from jax import lax
from jax.experimental import pallas as pl
from jax.experimental.pallas import tpu as pltpu
from jax.experimental.pallas import tpu_sc as plsc

BATCH = 4096
SEQ = 200
EMBED_DIM = 64
VOCAB = 1000000
NUM_WORKERS = 32
NB = BATCH // NUM_WORKERS
NG = NB // 16


def _body(xt_hbm, pairs_hbm, pos_hbm, out_hbm, idx_v, idx2_v, rows_v, out_t, pos_v, si, sg, sw):
    w = lax.axis_index("s") * 2 + lax.axis_index("c")
    col0 = NB * w
    iota = lax.iota(jnp.int32, 16)

    pltpu.sync_copy(pos_hbm, pos_v)

    def fetch_idx(l, b):
        return pltpu.async_copy(xt_hbm.at[l, pl.ds(col0, NB)], idx_v[b], si[b])

    def wait_idx(l, b):
        pltpu.make_async_copy(
            xt_hbm.at[l, pl.ds(col0, NB)], idx_v[b], si[b]
        ).wait()

    def gather(b):
        return pltpu.async_copy(pairs_hbm.at[idx2_v[b]], rows_v[b], sg[b])

    def wait_gather(b):
        pltpu.make_async_copy(pairs_hbm.at[idx2_v[b]], rows_v[b], sg[b]).wait()

    def halve(b):
        for g in range(NG):
            vg = idx_v[b][pl.ds(16 * g, 16)]
            idx2_v[b][pl.ds(16 * g, 16)] = lax.shift_right_logical(vg, 1)

    def write_out(l, b):
        return pltpu.async_copy(
            out_t[b],
            out_hbm.at[pl.ds(l * EMBED_DIM, EMBED_DIM), pl.ds(col0, NB)],
            sw[b],
        )

    def wait_write(l, b):
        pltpu.make_async_copy(
            out_t[b],
            out_hbm.at[pl.ds(l * EMBED_DIM, EMBED_DIM), pl.ds(col0, NB)],
            sw[b],
        ).wait()

    fetch_idx(0, 0).wait()
    halve(0)
    gather(0)
    fetch_idx(1, 1)

    def step(l, b, nb_):
        @pl.when(l + 1 < SEQ)
        def _():
            wait_idx(l + 1, nb_)
            halve(nb_)
            gather(nb_)

        wait_gather(b)

        rowg = [iota + 16 * g for g in range(NG)]
        baseg = [
            (idx_v[b][pl.ds(16 * g, 16)] & 1) * EMBED_DIM for g in range(NG)
        ]

        @pl.when(l + 2 < SEQ)
        def _():
            fetch_idx(l + 2, b)

        @pl.when(l >= 2)
        def _():
            wait_write(l - 2, b)

        lvec = jnp.broadcast_to(l, (16,)).astype(jnp.int32)

        def c_body(c, carry2):
            diag = (c + iota) & 15
            for k in range(EMBED_DIM // 16):
                dvec = 16 * k + diag
                p = plsc.load_gather(pos_v, [lvec, dvec])
                for g in range(NG):
                    v = plsc.load_gather(rows_v[b], [rowg[g], baseg[g] + dvec])
                    plsc.store_scatter(out_t[b], [dvec, rowg[g]], v + p)
            return carry2

        lax.fori_loop(0, 16, c_body, 0)
        write_out(l, b)

    def pair_body(i, carry):
        l = 2 * i
        step(l, 0, 1)
        step(l + 1, 1, 0)
        return carry

    lax.fori_loop(0, SEQ // 2, pair_body, 0)

    wait_write(SEQ - 2, 0)
    wait_write(SEQ - 1, 1)


N_BLK = VOCAB // 256
TAIL_START = N_BLK * 256
BLK_PER_TILE = 123


def _tbody(emb_hbm, pairs_hbm, in_v0, in_v1, out_v0, out_v1, tail_in, sr0, sr1, sw0, sw1):
    w = lax.axis_index("s") * 2 + lax.axis_index("c")
    iota = lax.iota(jnp.int32, 16)
    in_v = (in_v0, in_v1)
    out_v = (out_v0, out_v1)
    sr = (sr0, sr1)
    sw = (sw0, sw1)

    def blk_of(s):
        return w + 32 * s

    def start_of(blk):
        return pl.multiple_of(blk * 256, 256)

    def row0_of(blk):
        return pl.multiple_of(blk * 128, 128)

    def read(s, b):
        st = start_of(blk_of(s))
        return pltpu.async_copy(emb_hbm.at[pl.ds(st, 256), :], in_v[b], sr[b])

    def wait_read(s, b):
        st = start_of(blk_of(s))
        pltpu.make_async_copy(
            emb_hbm.at[pl.ds(st, 256), :], in_v[b], sr[b]
        ).wait()

    def write(s, b):
        r0 = row0_of(blk_of(s))
        return pltpu.async_copy(
            out_v[b], pairs_hbm.at[pl.ds(r0, 128), :], sw[b]
        )

    def wait_write(s, b):
        r0 = row0_of(blk_of(s))
        pltpu.make_async_copy(
            out_v[b], pairs_hbm.at[pl.ds(r0, 128), :], sw[b]
        ).wait()

    def repack(b, src, npairs):
        def p_body(p, carry):
            for q in range(2):
                for m in range(4):
                    sl = pl.ds(16 * m, 16)
                    out_v[b][p, pl.ds(64 * q + 16 * m, 16)] = src[2 * p + q, sl]
            return carry

        lax.fori_loop(0, npairs, p_body, 0, unroll=2)

    def compute(b):
        repack(b, in_v[b], 128)

    read(0, 0)

    def pair_body(jl, carry):
        for b in range(2):
            s = 2 * jl + b
            valid = blk_of(s) < N_BLK
            valid_next = blk_of(s + 1) < N_BLK

            @pl.when(valid_next)
            def _():
                read(s + 1, 1 - b)

            @pl.when(valid)
            def _():
                wait_read(s, b)

                @pl.when(s >= 2)
                def _():
                    wait_write(s - 2, b)

                compute(b)
                write(s, b)

        return carry

    lax.fori_loop(0, (BLK_PER_TILE + 1) // 2, pair_body, 0)

    @pl.when(blk_of(BLK_PER_TILE - 1) < N_BLK)
    def _():
        wait_write(BLK_PER_TILE - 1, 0)

    @pl.when(blk_of(BLK_PER_TILE - 1) >= N_BLK)
    def _():
        wait_write(BLK_PER_TILE - 3, 0)

    wait_write(BLK_PER_TILE - 2, 1)

    @pl.when(w == 31)
    def _():
        pltpu.async_copy(
            emb_hbm.at[pl.ds(TAIL_START, 64), :], tail_in, sr0
        ).wait()
        repack(0, tail_in, 32)
        pltpu.async_copy(
            out_v0.at[pl.ds(0, 32)],
            pairs_hbm.at[pl.ds(TAIL_START // 2, 32), :],
            sw0,
        ).wait()


def _wrapped_body(xt_hbm, pairs_hbm, pos_hbm, out_hbm,
                  idx_v0, idx_v1, idx2_v0, idx2_v1, rows_v0, rows_v1,
                  out_t0, out_t1, pos_v, si0, si1, sg0, sg1, sw0, sw1):
    _body(
        xt_hbm, pairs_hbm, pos_hbm, out_hbm,
        (idx_v0, idx_v1), (idx2_v0, idx2_v1), (rows_v0, rows_v1),
        (out_t0, out_t1), pos_v, (si0, si1), (sg0, sg1), (sw0, sw1),
    )


@jax.jit
def kernel(x, emb_table, pos_table):
    xt = x.T
    mesh = plsc.VectorSubcoreMesh(core_axis_name="c", subcore_axis_name="s")
    transpose_run = pl.kernel(
        _tbody,
        out_type=jax.ShapeDtypeStruct((VOCAB // 2, 2 * EMBED_DIM), jnp.float32),
        mesh=mesh,
        scratch_types=[
            pltpu.VMEM((256, EMBED_DIM), jnp.float32),
            pltpu.VMEM((256, EMBED_DIM), jnp.float32),
            pltpu.VMEM((128, 2 * EMBED_DIM), jnp.float32),
            pltpu.VMEM((128, 2 * EMBED_DIM), jnp.float32),
            pltpu.VMEM((64, EMBED_DIM), jnp.float32),
            pltpu.SemaphoreType.DMA,
            pltpu.SemaphoreType.DMA,
            pltpu.SemaphoreType.DMA,
            pltpu.SemaphoreType.DMA,
        ],
        compiler_params=pltpu.CompilerParams(needs_layout_passes=False),
    )
    pairs = transpose_run(emb_table)
    run = pl.kernel(
        _wrapped_body,
        out_type=jax.ShapeDtypeStruct((SEQ * EMBED_DIM, BATCH), jnp.float32),
        mesh=mesh,
        scratch_types=[
            pltpu.VMEM((NB,), jnp.int32),
            pltpu.VMEM((NB,), jnp.int32),
            pltpu.VMEM((NB,), jnp.int32),
            pltpu.VMEM((NB,), jnp.int32),
            pltpu.VMEM((NB, 2 * EMBED_DIM), jnp.float32),
            pltpu.VMEM((NB, 2 * EMBED_DIM), jnp.float32),
            pltpu.VMEM((EMBED_DIM, NB), jnp.float32),
            pltpu.VMEM((EMBED_DIM, NB), jnp.float32),
            pltpu.VMEM((SEQ, EMBED_DIM), jnp.float32),
            pltpu.SemaphoreType.DMA,
            pltpu.SemaphoreType.DMA,
            pltpu.SemaphoreType.DMA,
            pltpu.SemaphoreType.DMA,
            pltpu.SemaphoreType.DMA,
            pltpu.SemaphoreType.DMA,
        ],
        compiler_params=pltpu.CompilerParams(needs_layout_passes=False),
    )
    out2d = run(xt, pairs, pos_table)
    return out2d.reshape(SEQ, EMBED_DIM, BATCH).transpose(2, 0, 1)


# speedup vs baseline: 1.1632x; 1.1632x over previous
import jax
import jax.numpy as jnp
from jax import lax
from jax.experimental import pallas as pl
from jax.experimental.pallas import tpu as pltpu
from jax.experimental.pallas import tpu_sc as plsc

BATCH = 4096
SEQ = 200
EMBED_DIM = 64
VOCAB = 1000000
NUM_WORKERS = 32
NB = BATCH // NUM_WORKERS
NG = NB // 16


def _body(xt_hbm, pairs_hbm, pos_hbm, out_hbm, idx_v, idx2_v, rows_v, out_t, pos_v, si, sg, sw):
    w = lax.axis_index("s") * 2 + lax.axis_index("c")
    col0 = NB * w
    iota = lax.iota(jnp.int32, 16)

    pltpu.sync_copy(pos_hbm, pos_v)

    def fetch_idx(l, b):
        return pltpu.async_copy(xt_hbm.at[l, pl.ds(col0, NB)], idx_v[b], si[b])

    def wait_idx(l, b):
        pltpu.make_async_copy(
            xt_hbm.at[l, pl.ds(col0, NB)], idx_v[b], si[b]
        ).wait()

    def gather(b):
        return pltpu.async_copy(pairs_hbm.at[idx2_v[b]], rows_v[b], sg[b])

    def wait_gather(b):
        pltpu.make_async_copy(pairs_hbm.at[idx2_v[b]], rows_v[b], sg[b]).wait()

    def halve(b):
        for g in range(NG):
            vg = idx_v[b][pl.ds(16 * g, 16)]
            idx2_v[b][pl.ds(16 * g, 16)] = lax.shift_right_logical(vg, 1)

    def write_out(l, b):
        return pltpu.async_copy(
            out_t[b],
            out_hbm.at[pl.ds(l * EMBED_DIM, EMBED_DIM), pl.ds(col0, NB)],
            sw[b],
        )

    def wait_write(l, b):
        pltpu.make_async_copy(
            out_t[b],
            out_hbm.at[pl.ds(l * EMBED_DIM, EMBED_DIM), pl.ds(col0, NB)],
            sw[b],
        ).wait()

    fetch_idx(0, 0).wait()
    halve(0)
    gather(0)
    fetch_idx(1, 1)

    def step(l, b, nb_):
        @pl.when(l + 1 < SEQ)
        def _():
            wait_idx(l + 1, nb_)
            halve(nb_)
            gather(nb_)

        wait_gather(b)

        rowg = [iota + 16 * g for g in range(NG)]
        baseg = [
            (idx_v[b][pl.ds(16 * g, 16)] & 1) * EMBED_DIM for g in range(NG)
        ]

        @pl.when(l + 2 < SEQ)
        def _():
            fetch_idx(l + 2, b)

        @pl.when(l >= 2)
        def _():
            wait_write(l - 2, b)

        lvec = jnp.broadcast_to(l, (16,)).astype(jnp.int32)

        def c_body(c, carry2):
            diag = (c + iota) & 15
            for k in range(EMBED_DIM // 16):
                dvec = 16 * k + diag
                p = plsc.load_gather(pos_v, [lvec, dvec])
                for g in range(NG):
                    v = plsc.load_gather(rows_v[b], [rowg[g], baseg[g] + dvec])
                    plsc.store_scatter(out_t[b], [dvec, rowg[g]], v + p)
            return carry2

        lax.fori_loop(0, 16, c_body, 0)
        write_out(l, b)

    def pair_body(i, carry):
        l = 2 * i
        step(l, 0, 1)
        step(l + 1, 1, 0)
        return carry

    lax.fori_loop(0, SEQ // 2, pair_body, 0)

    wait_write(SEQ - 2, 0)
    wait_write(SEQ - 1, 1)


def _wrapped_body(xt_hbm, pairs_hbm, pos_hbm, out_hbm,
                  idx_v0, idx_v1, idx2_v0, idx2_v1, rows_v0, rows_v1,
                  out_t0, out_t1, pos_v, si0, si1, sg0, sg1, sw0, sw1):
    _body(
        xt_hbm, pairs_hbm, pos_hbm, out_hbm,
        (idx_v0, idx_v1), (idx2_v0, idx2_v1), (rows_v0, rows_v1),
        (out_t0, out_t1), pos_v, (si0, si1), (sg0, sg1), (sw0, sw1),
    )


@jax.jit
def kernel(x, emb_table, pos_table):
    xt = x.T
    pairs = emb_table.reshape(VOCAB // 2, 2 * EMBED_DIM)
    mesh = plsc.VectorSubcoreMesh(core_axis_name="c", subcore_axis_name="s")
    run = pl.kernel(
        _wrapped_body,
        out_type=jax.ShapeDtypeStruct((SEQ * EMBED_DIM, BATCH), jnp.float32),
        mesh=mesh,
        scratch_types=[
            pltpu.VMEM((NB,), jnp.int32),
            pltpu.VMEM((NB,), jnp.int32),
            pltpu.VMEM((NB,), jnp.int32),
            pltpu.VMEM((NB,), jnp.int32),
            pltpu.VMEM((NB, 2 * EMBED_DIM), jnp.float32),
            pltpu.VMEM((NB, 2 * EMBED_DIM), jnp.float32),
            pltpu.VMEM((EMBED_DIM, NB), jnp.float32),
            pltpu.VMEM((EMBED_DIM, NB), jnp.float32),
            pltpu.VMEM((SEQ, EMBED_DIM), jnp.float32),
            pltpu.SemaphoreType.DMA,
            pltpu.SemaphoreType.DMA,
            pltpu.SemaphoreType.DMA,
            pltpu.SemaphoreType.DMA,
            pltpu.SemaphoreType.DMA,
            pltpu.SemaphoreType.DMA,
        ],
        compiler_params=pltpu.CompilerParams(needs_layout_passes=False),
    )
    out2d = run(xt, pairs, pos_table)
    return out2d.reshape(SEQ, EMBED_DIM, BATCH).transpose(2, 0, 1)
